# baseline (speedup 1.0000x reference)
import functools

import jax
import jax.numpy as jnp
from jax import lax
from jax.experimental import pallas as pl
from jax.experimental.pallas import tpu as pltpu
from jax.experimental.pallas import tpu_sc as plsc

NC, NS = 2, 16
NW = NC * NS
SENTINEL = 0x3FFFFFFF

_sc_mesh = lambda: plsc.VectorSubcoreMesh(
    core_axis_name="c", subcore_axis_name="s", num_cores=NC, num_subcores=NS
)
_sc_params = lambda: pltpu.CompilerParams(needs_layout_passes=False)


def _make_deg_kernel(EP, NHR):
    B = 2000
    ept = EP // NW
    nblk = ept // B
    NHALF = 2
    NHW = NHR * 128
    HW = NHW // NHALF
    wpt = HW // 16

    @functools.partial(
        pl.kernel,
        out_type=jax.ShapeDtypeStruct((2 * NHW,), jnp.float32),
        mesh=_sc_mesh(),
        scratch_types=[
            pltpu.VMEM_SHARED((NS, HW), jnp.float32),
            pltpu.VMEM((HW,), jnp.float32),
            pltpu.VMEM((wpt,), jnp.float32),
            pltpu.VMEM((wpt,), jnp.float32),
            pltpu.VMEM((B,), jnp.int32),
            pltpu.SemaphoreType.DMA,
        ],
        compiler_params=_sc_params(),
    )
    def k(dst, out, parts, hist, accv, pbuf, dv, sem):
        core = lax.axis_index("c")
        sid = lax.axis_index("s")
        wid = sid * NC + core
        ones = jnp.ones((16,), jnp.float32)

        for h in range(NHALF):
            lo = h * HW

            def zh(i, _):
                hist[pl.ds(i * 16, 16)] = jnp.zeros((16,), jnp.float32)
                return 0

            lax.fori_loop(0, HW // 16, zh, 0)

            def blk_body(b, _):
                pltpu.sync_copy(dst.at[pl.ds(wid * ept + b * B, B)], dv)

                def g_body(g, _):
                    i16 = dv[pl.ds(g * 16, 16)] - lo
                    m = (i16 >= 0) & (i16 < HW)
                    plsc.addupdate_scatter(hist, [i16], ones, mask=m)
                    return 0

                lax.fori_loop(0, B // 16, g_body, 0)
                return 0

            lax.fori_loop(0, nblk, blk_body, 0)
            pltpu.sync_copy(hist, parts.at[sid])
            plsc.subcore_barrier()
            w0 = sid * wpt

            def za(i, _):
                accv[pl.ds(i * 16, 16)] = jnp.zeros((16,), jnp.float32)
                return 0

            lax.fori_loop(0, wpt // 16, za, 0)
            for t in range(NS):
                pltpu.sync_copy(parts.at[t].at[pl.ds(w0, wpt)], pbuf)

                def addb(i, _):
                    accv[pl.ds(i * 16, 16)] = (
                        accv[pl.ds(i * 16, 16)] + pbuf[pl.ds(i * 16, 16)]
                    )
                    return 0

                lax.fori_loop(0, wpt // 16, addb, 0)
            pltpu.sync_copy(
                accv, out.at[pl.ds(core * NHW + h * HW + w0, wpt)]
            )
            plsc.subcore_barrier()

    return k


def _make_scatter_kernel(NV, EP, C, K, n_val):
    B = 2000
    ept = EP // 16
    nblk = ept // B
    ngmax = (B + 127) // 128
    ctile = C // 16
    kpc = K // 2
    nring = 2 // n_val

    out_t = tuple(
        jax.ShapeDtypeStruct((K * C, 128), jnp.float32) for _ in range(n_val)
    )
    scratch = [
        [pltpu.VMEM_SHARED((C + 16, 128), jnp.float32) for _ in range(n_val)],
        pltpu.VMEM((32, 128), jnp.float32),
        pltpu.VMEM((B,), jnp.int32),
        pltpu.VMEM((B,), jnp.int32),
        pltpu.VMEM((B + 144,), jnp.int32),
        pltpu.VMEM((B + 144,), jnp.int32),
        [pltpu.VMEM((128,), jnp.int32) for _ in range(nring)],
        [
            [pltpu.VMEM((128, 128), jnp.float32) for _ in range(n_val)]
            for _ in range(nring)
        ],
        pltpu.SemaphoreType.DMA,
        pltpu.SemaphoreType.DMA,
    ]

    @functools.partial(
        pl.kernel,
        out_type=out_t,
        mesh=_sc_mesh(),
        scratch_types=scratch,
        compiler_params=_sc_params(),
    )
    def k(*args):
        ys = args[:n_val]
        src, dst = args[n_val], args[n_val + 1]
        outs = args[n_val + 2 : 2 * n_val + 2]
        (slabs, zbuf, dv, sv, cd, cs, idxr, rowr, semg, sems) = args[
            2 * n_val + 2 :
        ]
        core = lax.axis_index("c")
        sid = lax.axis_index("s")

        def zb(i, _):
            zbuf[i // 8, pl.ds((i % 8) * 16, 16)] = jnp.zeros((16,), jnp.float32)
            return 0

        lax.fori_loop(0, 32 * 8, zb, 0)

        def chunk_body(ci, _):
            chunk = ci * 2 + core
            lo = chunk * C

            def zc(i, _):
                for v in range(n_val):
                    pltpu.sync_copy(
                        zbuf, slabs[v].at[pl.ds(sid * ctile + i * 32, 32)]
                    )
                return 0

            lax.fori_loop(0, ctile // 32, zc, 0)

            @pl.when(sid == 0)
            def _():
                for v in range(n_val):
                    pltpu.sync_copy(
                        zbuf.at[pl.ds(0, 16)], slabs[v].at[pl.ds(C, 16)]
                    )

            plsc.subcore_barrier()

            def blk_body(b, _):
                base = sid * ept + b * B
                pltpu.sync_copy(dst.at[pl.ds(base, B)], dv)
                pltpu.sync_copy(src.at[pl.ds(base, B)], sv)

                def g_body(g, cnt):
                    d16 = dv[pl.ds(g * 16, 16)]
                    s16 = sv[pl.ds(g * 16, 16)]
                    m = (d16 >= lo) & (d16 < lo + C)
                    plsc.store_compressed(cd.at[pl.ds(cnt, 16)], d16 - lo, mask=m)
                    plsc.store_compressed(cs.at[pl.ds(cnt, 16)], s16, mask=m)
                    return cnt + jnp.sum(m.astype(jnp.int32))

                cnt = lax.fori_loop(0, B // 16, g_body, 0)
                trash = jnp.full((16,), C, jnp.int32)
                zero16 = jnp.zeros((16,), jnp.int32)

                def pb(p, _):
                    cd[pl.ds(cnt + p * 16, 16)] = trash
                    cs[pl.ds(cnt + p * 16, 16)] = zero16
                    return 0

                lax.fori_loop(0, 8, pb, 0)
                ng = (cnt + 127) // 128
                for j in range(ngmax):
                    r = j % nring

                    @pl.when(j < ng)
                    def _():
                        for q in range(8):
                            idxr[r][pl.ds(q * 16, 16)] = cd[
                                pl.ds(j * 128 + q * 16, 16)
                            ]
                        for v in range(n_val):
                            pltpu.async_copy(
                                ys[v].at[cs.at[pl.ds(j * 128, 128)]],
                                rowr[r][v],
                                semg,
                            ).wait()
                        for v in range(n_val):
                            pltpu.async_copy(
                                rowr[r][v], slabs[v].at[idxr[r]], sems, add=True
                            ).wait()

                return 0

            lax.fori_loop(0, nblk, blk_body, 0)
            plsc.subcore_barrier()
            for v in range(n_val):
                pltpu.sync_copy(
                    slabs[v].at[pl.ds(sid * ctile, ctile)],
                    outs[v].at[pl.ds(chunk * C + sid * ctile, ctile)],
                )
            plsc.subcore_barrier()
            return 0

        lax.fori_loop(0, kpc, chunk_body, 0)

    return k


def _make_prescale(N, BT):
    grid = (pl.cdiv(N, BT),)

    def body(deg_ref, x_ref, y_ref, d_ref):
        d = lax.rsqrt(deg_ref[...] + 1.0)
        d_ref[...] = d
        y_ref[...] = x_ref[...] * d.reshape(BT, 1)

    return pl.pallas_call(
        body,
        grid=grid,
        in_specs=[
            pl.BlockSpec((1, BT), lambda i: (0, i)),
            pl.BlockSpec((BT, 128), lambda i: (i, 0)),
        ],
        out_specs=[
            pl.BlockSpec((BT, 128), lambda i: (i, 0)),
            pl.BlockSpec((1, BT), lambda i: (0, i)),
        ],
        out_shape=[
            jax.ShapeDtypeStruct((N, 128), jnp.float32),
            jax.ShapeDtypeStruct((1, N), jnp.float32),
        ],
    )


def _make_layer0(N, BT):
    grid = (pl.cdiv(N, BT),)

    def body(s_ref, y_ref, d_ref, w_ref, b_ref, a_ref, b2_ref):
        d = d_ref[...].reshape(BT, 1)
        a = (s_ref[...] + y_ref[...]) * d
        z = jnp.maximum(
            jnp.dot(a, w_ref[...], preferred_element_type=jnp.float32)
            + b_ref[...],
            0.0,
        )
        y1 = z * d
        a_ref[...] = y1[:, :128]
        b2_ref[...] = y1[:, 128:]

    return pl.pallas_call(
        body,
        grid=grid,
        in_specs=[
            pl.BlockSpec((BT, 128), lambda i: (i, 0)),
            pl.BlockSpec((BT, 128), lambda i: (i, 0)),
            pl.BlockSpec((1, BT), lambda i: (0, i)),
            pl.BlockSpec((128, 256), lambda i: (0, 0)),
            pl.BlockSpec((1, 256), lambda i: (0, 0)),
        ],
        out_specs=[
            pl.BlockSpec((BT, 128), lambda i: (i, 0)),
            pl.BlockSpec((BT, 128), lambda i: (i, 0)),
        ],
        out_shape=[
            jax.ShapeDtypeStruct((N, 128), jnp.float32),
            jax.ShapeDtypeStruct((N, 128), jnp.float32),
        ],
    )


def _make_layer12(N, BT):
    grid = (pl.cdiv(N, BT),)

    def body(sa, sb, ya, yb, d_ref, w1, b1, w2, out):
        d = d_ref[...].reshape(BT, 1)
        a = (
            jnp.concatenate([sa[...] + ya[...], sb[...] + yb[...]], axis=1) * d
        )
        h = jnp.maximum(
            jnp.dot(a, w1[...], preferred_element_type=jnp.float32) + b1[...],
            0.0,
        )
        out[...] = jnp.dot(h, w2[...], preferred_element_type=jnp.float32) * d

    return pl.pallas_call(
        body,
        grid=grid,
        in_specs=[
            pl.BlockSpec((BT, 128), lambda i: (i, 0)),
            pl.BlockSpec((BT, 128), lambda i: (i, 0)),
            pl.BlockSpec((BT, 128), lambda i: (i, 0)),
            pl.BlockSpec((BT, 128), lambda i: (i, 0)),
            pl.BlockSpec((1, BT), lambda i: (0, i)),
            pl.BlockSpec((256, 512), lambda i: (0, 0)),
            pl.BlockSpec((1, 512), lambda i: (0, 0)),
            pl.BlockSpec((512, 128), lambda i: (0, 0)),
        ],
        out_specs=pl.BlockSpec((BT, 128), lambda i: (i, 0)),
        out_shape=jax.ShapeDtypeStruct((N, 128), jnp.float32),
    )


def _make_final(N, BT):
    grid = (pl.cdiv(N, BT),)

    def body(s_ref, y_ref, d_ref, b_ref, out):
        d = d_ref[...].reshape(BT, 1)
        out[...] = (s_ref[...] + y_ref[...]) * d + b_ref[...]

    return pl.pallas_call(
        body,
        grid=grid,
        in_specs=[
            pl.BlockSpec((BT, 128), lambda i: (i, 0)),
            pl.BlockSpec((BT, 128), lambda i: (i, 0)),
            pl.BlockSpec((1, BT), lambda i: (0, i)),
            pl.BlockSpec((1, 128), lambda i: (0, 0)),
        ],
        out_specs=pl.BlockSpec((BT, 128), lambda i: (i, 0)),
        out_shape=jax.ShapeDtypeStruct((N, 128), jnp.float32),
    )


def kernel(x, edge_index, W0, b0, W1, b1, W2, b2):
    N = x.shape[0]
    E = edge_index.shape[1]

    src = edge_index[0].astype(jnp.int32)
    dst = edge_index[1].astype(jnp.int32)
    EP = pl.cdiv(E, 64000) * 64000
    if EP != E:
        pad = jnp.full((EP - E,), SENTINEL, jnp.int32)
        src = jnp.concatenate([src, jnp.zeros((EP - E,), jnp.int32)])
        dst = jnp.concatenate([dst, pad])

    NHR = pl.cdiv(N, 128 * 128) * 128
    degp = _make_deg_kernel(EP, NHR)(dst)
    dd = degp.reshape(2, NHR * 128)
    deg = (dd[0, :N] + dd[1, :N]).reshape(1, N)

    BT = 2048
    y0, dcol = _make_prescale(N, BT)(deg, x)

    C1, K1 = 10240, 2 * pl.cdiv(N, 2 * 10240)
    scat1 = _make_scatter_kernel(N, EP, C1, K1, 1)
    (s0,) = scat1(y0, src, dst)
    y1a, y1b = _make_layer0(N, BT)(s0, y0, dcol, W0, b0.reshape(1, -1))

    C2, K2 = 4608, 2 * pl.cdiv(N, 2 * 4608)
    scat2 = _make_scatter_kernel(N, EP, C2, K2, 2)
    s1a, s1b = scat2(y1a, y1b, src, dst)
    y2 = _make_layer12(N, BT)(
        s1a, s1b, y1a, y1b, dcol, W1, b1.reshape(1, -1), W2
    )

    (s2,) = scat1(y2, src, dst)
    out = _make_final(N, BT)(s2, y2, dcol, b2.reshape(1, -1))
    return out


# speedup vs baseline: 5.9350x; 5.9350x over previous
import functools

import jax
import jax.numpy as jnp
from jax import lax
from jax.experimental import pallas as pl
from jax.experimental.pallas import tpu as pltpu
from jax.experimental.pallas import tpu_sc as plsc

NC, NS = 2, 16
NW = NC * NS
SENTINEL = 0x3FFFFFFF

_sc_mesh = lambda: plsc.VectorSubcoreMesh(
    core_axis_name="c", subcore_axis_name="s", num_cores=NC, num_subcores=NS
)
_sc_params = lambda: pltpu.CompilerParams(needs_layout_passes=False)


def _make_deg_kernel(EP, NHR):
    B = 2000
    ept = EP // NW
    nblk = ept // B
    NHALF = 2
    NHW = NHR * 128
    HW = NHW // NHALF
    wpt = HW // 16

    @functools.partial(
        pl.kernel,
        out_type=jax.ShapeDtypeStruct((2 * NHW,), jnp.float32),
        mesh=_sc_mesh(),
        scratch_types=[
            pltpu.VMEM_SHARED((NS, HW), jnp.float32),
            pltpu.VMEM((HW,), jnp.float32),
            pltpu.VMEM((wpt,), jnp.float32),
            pltpu.VMEM((wpt,), jnp.float32),
            pltpu.VMEM((B,), jnp.int32),
            pltpu.SemaphoreType.DMA,
        ],
        compiler_params=_sc_params(),
    )
    def k(dst, out, parts, hist, accv, pbuf, dv, sem):
        core = lax.axis_index("c")
        sid = lax.axis_index("s")
        wid = sid * NC + core
        ones = jnp.ones((16,), jnp.float32)

        for h in range(NHALF):
            lo = h * HW

            def zh(i, _):
                hist[pl.ds(i * 16, 16)] = jnp.zeros((16,), jnp.float32)
                return 0

            lax.fori_loop(0, HW // 16, zh, 0)

            def blk_body(b, _):
                pltpu.sync_copy(dst.at[pl.ds(wid * ept + b * B, B)], dv)

                def g_body(g, _):
                    i16 = dv[pl.ds(g * 16, 16)] - lo
                    m = (i16 >= 0) & (i16 < HW)
                    plsc.addupdate_scatter(hist, [i16], ones, mask=m)
                    return 0

                lax.fori_loop(0, B // 16, g_body, 0)
                return 0

            lax.fori_loop(0, nblk, blk_body, 0)
            pltpu.sync_copy(hist, parts.at[sid])
            plsc.subcore_barrier()
            w0 = sid * wpt

            def za(i, _):
                accv[pl.ds(i * 16, 16)] = jnp.zeros((16,), jnp.float32)
                return 0

            lax.fori_loop(0, wpt // 16, za, 0)
            for t in range(NS):
                pltpu.sync_copy(parts.at[t].at[pl.ds(w0, wpt)], pbuf)

                def addb(i, _):
                    accv[pl.ds(i * 16, 16)] = (
                        accv[pl.ds(i * 16, 16)] + pbuf[pl.ds(i * 16, 16)]
                    )
                    return 0

                lax.fori_loop(0, wpt // 16, addb, 0)
            pltpu.sync_copy(
                accv, out.at[pl.ds(core * NHW + h * HW + w0, wpt)]
            )
            plsc.subcore_barrier()

    return k


def _make_scatter_kernel(NV, EP, C, K, n_val):
    B = 2000
    ept = EP // 16
    nblk = ept // B
    ctile = C // 16
    kpc = K // 2
    KR = 8 // n_val

    out_t = tuple(
        jax.ShapeDtypeStruct((K * C, 128), jnp.float32) for _ in range(n_val)
    )
    scratch = [
        [pltpu.VMEM_SHARED((C + 16, 128), jnp.float32) for _ in range(n_val)],
        pltpu.VMEM((32, 128), jnp.float32),
        pltpu.VMEM((B,), jnp.int32),
        pltpu.VMEM((B,), jnp.int32),
        pltpu.VMEM((B + 32,), jnp.int32),
        pltpu.VMEM((B + 32,), jnp.int32),
        [
            [pltpu.VMEM((16, 128), jnp.float32) for _ in range(n_val)]
            for _ in range(KR)
        ],
        [pltpu.SemaphoreType.DMA for _ in range(KR)],
        [pltpu.SemaphoreType.DMA for _ in range(KR)],
    ]

    @functools.partial(
        pl.kernel,
        out_type=out_t,
        mesh=_sc_mesh(),
        scratch_types=scratch,
        compiler_params=_sc_params(),
    )
    def k(*args):
        ys = args[:n_val]
        src, dst = args[n_val], args[n_val + 1]
        outs = args[n_val + 2 : 2 * n_val + 2]
        (slabs, zbuf, dv, sv, cd, cs, rowr, semg, sems) = args[
            2 * n_val + 2 :
        ]
        core = lax.axis_index("c")
        sid = lax.axis_index("s")

        def zb(i, _):
            zbuf[i // 8, pl.ds((i % 8) * 16, 16)] = jnp.zeros((16,), jnp.float32)
            return 0

        lax.fori_loop(0, 32 * 8, zb, 0)

        def chunk_body(ci, _):
            chunk = ci * 2 + core
            lo = chunk * C

            def zc(i, _):
                for v in range(n_val):
                    pltpu.sync_copy(
                        zbuf, slabs[v].at[pl.ds(sid * ctile + i * 32, 32)]
                    )
                return 0

            lax.fori_loop(0, ctile // 32, zc, 0)

            @pl.when(sid == 0)
            def _():
                for v in range(n_val):
                    pltpu.sync_copy(
                        zbuf.at[pl.ds(0, 16)], slabs[v].at[pl.ds(C, 16)]
                    )

            plsc.subcore_barrier()

            def blk_body(b, _):
                base = sid * ept + b * B
                pltpu.sync_copy(dst.at[pl.ds(base, B)], dv)
                pltpu.sync_copy(src.at[pl.ds(base, B)], sv)

                def g_body(g, cnt):
                    d16 = dv[pl.ds(g * 16, 16)]
                    s16 = sv[pl.ds(g * 16, 16)]
                    m = (d16 >= lo) & (d16 < lo + C)
                    plsc.store_compressed(cd.at[pl.ds(cnt, 16)], d16 - lo, mask=m)
                    plsc.store_compressed(cs.at[pl.ds(cnt, 16)], s16, mask=m)
                    return cnt + jnp.sum(m.astype(jnp.int32))

                cnt = lax.fori_loop(0, B // 16, g_body, 0)
                cd[pl.ds(cnt, 16)] = jnp.full((16,), C, jnp.int32)
                cs[pl.ds(cnt, 16)] = jnp.zeros((16,), jnp.int32)
                ngv = (cnt + 15) // 16
                nround = (ngv + KR - 1) // KR

                def r_body(i, _):
                    g0 = i * KR
                    for r in range(KR):

                        @pl.when(g0 + r < ngv)
                        def _():
                            s16 = cs[pl.ds((g0 + r) * 16, 16)]
                            for v in range(n_val):
                                pltpu.async_copy(
                                    ys[v].at[s16], rowr[r][v], semg[r]
                                )
                    for r in range(KR):

                        @pl.when(g0 + r < ngv)
                        def _():
                            d16 = cd[pl.ds((g0 + r) * 16, 16)]
                            for v in range(n_val):
                                pltpu.make_async_copy(
                                    ys[v].at[d16], rowr[r][v], semg[r]
                                ).wait()
                                pltpu.async_copy(
                                    rowr[r][v], slabs[v].at[d16], sems[r],
                                    add=True,
                                ).wait()

                    return 0

                lax.fori_loop(0, nround, r_body, 0)
                return 0

            lax.fori_loop(0, nblk, blk_body, 0)
            plsc.subcore_barrier()
            for v in range(n_val):
                pltpu.sync_copy(
                    slabs[v].at[pl.ds(sid * ctile, ctile)],
                    outs[v].at[pl.ds(chunk * C + sid * ctile, ctile)],
                )
            plsc.subcore_barrier()
            return 0

        lax.fori_loop(0, kpc, chunk_body, 0)

    return k


def _make_prescale(N, BT):
    grid = (pl.cdiv(N, BT),)

    def body(deg_ref, x_ref, y_ref, d_ref):
        d = lax.rsqrt(deg_ref[...] + 1.0)
        d_ref[...] = d
        y_ref[...] = x_ref[...] * d.reshape(BT, 1)

    return pl.pallas_call(
        body,
        grid=grid,
        in_specs=[
            pl.BlockSpec((1, BT), lambda i: (0, i)),
            pl.BlockSpec((BT, 128), lambda i: (i, 0)),
        ],
        out_specs=[
            pl.BlockSpec((BT, 128), lambda i: (i, 0)),
            pl.BlockSpec((1, BT), lambda i: (0, i)),
        ],
        out_shape=[
            jax.ShapeDtypeStruct((N, 128), jnp.float32),
            jax.ShapeDtypeStruct((1, N), jnp.float32),
        ],
    )


def _make_layer0(N, BT):
    grid = (pl.cdiv(N, BT),)

    def body(s_ref, y_ref, d_ref, w_ref, b_ref, a_ref, b2_ref):
        d = d_ref[...].reshape(BT, 1)
        a = (s_ref[...] + y_ref[...]) * d
        z = jnp.maximum(
            jnp.dot(a, w_ref[...], preferred_element_type=jnp.float32)
            + b_ref[...],
            0.0,
        )
        y1 = z * d
        a_ref[...] = y1[:, :128]
        b2_ref[...] = y1[:, 128:]

    return pl.pallas_call(
        body,
        grid=grid,
        in_specs=[
            pl.BlockSpec((BT, 128), lambda i: (i, 0)),
            pl.BlockSpec((BT, 128), lambda i: (i, 0)),
            pl.BlockSpec((1, BT), lambda i: (0, i)),
            pl.BlockSpec((128, 256), lambda i: (0, 0)),
            pl.BlockSpec((1, 256), lambda i: (0, 0)),
        ],
        out_specs=[
            pl.BlockSpec((BT, 128), lambda i: (i, 0)),
            pl.BlockSpec((BT, 128), lambda i: (i, 0)),
        ],
        out_shape=[
            jax.ShapeDtypeStruct((N, 128), jnp.float32),
            jax.ShapeDtypeStruct((N, 128), jnp.float32),
        ],
    )


def _make_layer12(N, BT):
    grid = (pl.cdiv(N, BT),)

    def body(sa, sb, ya, yb, d_ref, w1, b1, w2, out):
        d = d_ref[...].reshape(BT, 1)
        a = (
            jnp.concatenate([sa[...] + ya[...], sb[...] + yb[...]], axis=1) * d
        )
        h = jnp.maximum(
            jnp.dot(a, w1[...], preferred_element_type=jnp.float32) + b1[...],
            0.0,
        )
        out[...] = jnp.dot(h, w2[...], preferred_element_type=jnp.float32) * d

    return pl.pallas_call(
        body,
        grid=grid,
        in_specs=[
            pl.BlockSpec((BT, 128), lambda i: (i, 0)),
            pl.BlockSpec((BT, 128), lambda i: (i, 0)),
            pl.BlockSpec((BT, 128), lambda i: (i, 0)),
            pl.BlockSpec((BT, 128), lambda i: (i, 0)),
            pl.BlockSpec((1, BT), lambda i: (0, i)),
            pl.BlockSpec((256, 512), lambda i: (0, 0)),
            pl.BlockSpec((1, 512), lambda i: (0, 0)),
            pl.BlockSpec((512, 128), lambda i: (0, 0)),
        ],
        out_specs=pl.BlockSpec((BT, 128), lambda i: (i, 0)),
        out_shape=jax.ShapeDtypeStruct((N, 128), jnp.float32),
    )


def _make_final(N, BT):
    grid = (pl.cdiv(N, BT),)

    def body(s_ref, y_ref, d_ref, b_ref, out):
        d = d_ref[...].reshape(BT, 1)
        out[...] = (s_ref[...] + y_ref[...]) * d + b_ref[...]

    return pl.pallas_call(
        body,
        grid=grid,
        in_specs=[
            pl.BlockSpec((BT, 128), lambda i: (i, 0)),
            pl.BlockSpec((BT, 128), lambda i: (i, 0)),
            pl.BlockSpec((1, BT), lambda i: (0, i)),
            pl.BlockSpec((1, 128), lambda i: (0, 0)),
        ],
        out_specs=pl.BlockSpec((BT, 128), lambda i: (i, 0)),
        out_shape=jax.ShapeDtypeStruct((N, 128), jnp.float32),
    )


def kernel(x, edge_index, W0, b0, W1, b1, W2, b2):
    N = x.shape[0]
    E = edge_index.shape[1]

    src = edge_index[0].astype(jnp.int32)
    dst = edge_index[1].astype(jnp.int32)
    EP = pl.cdiv(E, 64000) * 64000
    if EP != E:
        pad = jnp.full((EP - E,), SENTINEL, jnp.int32)
        src = jnp.concatenate([src, jnp.zeros((EP - E,), jnp.int32)])
        dst = jnp.concatenate([dst, pad])

    NHR = pl.cdiv(N, 128 * 128) * 128
    degp = _make_deg_kernel(EP, NHR)(dst)
    dd = degp.reshape(2, NHR * 128)
    deg = (dd[0, :N] + dd[1, :N]).reshape(1, N)

    BT = 2048
    y0, dcol = _make_prescale(N, BT)(deg, x)

    C1, K1 = 12288, 2 * pl.cdiv(N, 2 * 12288)
    scat1 = _make_scatter_kernel(N, EP, C1, K1, 1)
    (s0,) = scat1(y0, src, dst)
    y1a, y1b = _make_layer0(N, BT)(s0, y0, dcol, W0, b0.reshape(1, -1))

    C2, K2 = 5632, 2 * pl.cdiv(N, 2 * 5632)
    scat2 = _make_scatter_kernel(N, EP, C2, K2, 2)
    s1a, s1b = scat2(y1a, y1b, src, dst)
    y2 = _make_layer12(N, BT)(
        s1a, s1b, y1a, y1b, dcol, W1, b1.reshape(1, -1), W2
    )

    (s2,) = scat1(y2, src, dst)
    out = _make_final(N, BT)(s2, y2, dcol, b2.reshape(1, -1))
    return out


# speedup vs baseline: 5.9708x; 1.0060x over previous
import functools

import jax
import jax.numpy as jnp
from jax import lax
from jax.experimental import pallas as pl
from jax.experimental.pallas import tpu as pltpu
from jax.experimental.pallas import tpu_sc as plsc

NC, NS = 2, 16
NW = NC * NS
SENTINEL = 0x3FFFFFFF

_sc_mesh = lambda: plsc.VectorSubcoreMesh(
    core_axis_name="c", subcore_axis_name="s", num_cores=NC, num_subcores=NS
)
_sc_params = lambda: pltpu.CompilerParams(needs_layout_passes=False)


def _make_deg_kernel(EP, NHR):
    B = 2000
    ept = EP // NW
    nblk = ept // B
    NHALF = 2
    NHW = NHR * 128
    HW = NHW // NHALF
    wpt = HW // 16

    @functools.partial(
        pl.kernel,
        out_type=jax.ShapeDtypeStruct((2 * NHW,), jnp.float32),
        mesh=_sc_mesh(),
        scratch_types=[
            pltpu.VMEM_SHARED((NS, HW), jnp.float32),
            pltpu.VMEM((HW,), jnp.float32),
            pltpu.VMEM((wpt,), jnp.float32),
            pltpu.VMEM((wpt,), jnp.float32),
            pltpu.VMEM((B,), jnp.int32),
            pltpu.SemaphoreType.DMA,
        ],
        compiler_params=_sc_params(),
    )
    def k(dst, out, parts, hist, accv, pbuf, dv, sem):
        core = lax.axis_index("c")
        sid = lax.axis_index("s")
        wid = sid * NC + core
        ones = jnp.ones((16,), jnp.float32)

        for h in range(NHALF):
            lo = h * HW

            def zh(i, _):
                hist[pl.ds(i * 16, 16)] = jnp.zeros((16,), jnp.float32)
                return 0

            lax.fori_loop(0, HW // 16, zh, 0)

            def blk_body(b, _):
                pltpu.sync_copy(dst.at[pl.ds(wid * ept + b * B, B)], dv)

                def g_body(g, _):
                    i16 = dv[pl.ds(g * 16, 16)] - lo
                    m = (i16 >= 0) & (i16 < HW)
                    plsc.addupdate_scatter(hist, [i16], ones, mask=m)
                    return 0

                lax.fori_loop(0, B // 16, g_body, 0)
                return 0

            lax.fori_loop(0, nblk, blk_body, 0)
            pltpu.sync_copy(hist, parts.at[sid])
            plsc.subcore_barrier()
            w0 = sid * wpt

            def za(i, _):
                accv[pl.ds(i * 16, 16)] = jnp.zeros((16,), jnp.float32)
                return 0

            lax.fori_loop(0, wpt // 16, za, 0)
            for t in range(NS):
                pltpu.sync_copy(parts.at[t].at[pl.ds(w0, wpt)], pbuf)

                def addb(i, _):
                    accv[pl.ds(i * 16, 16)] = (
                        accv[pl.ds(i * 16, 16)] + pbuf[pl.ds(i * 16, 16)]
                    )
                    return 0

                lax.fori_loop(0, wpt // 16, addb, 0)
            pltpu.sync_copy(
                accv, out.at[pl.ds(core * NHW + h * HW + w0, wpt)]
            )
            plsc.subcore_barrier()

    return k


def _make_scatter_kernel(NV, EP, C, K, n_val):
    B = 2000
    ept = EP // 16
    nblk = ept // B
    ctile = C // 16
    kpc = K // 2
    KR = 8 // n_val

    out_t = tuple(
        jax.ShapeDtypeStruct((K * C, 128), jnp.float32) for _ in range(n_val)
    )
    scratch = [
        [pltpu.VMEM_SHARED((C + 16, 128), jnp.float32) for _ in range(n_val)],
        pltpu.VMEM((32, 128), jnp.float32),
        pltpu.VMEM((B,), jnp.int32),
        pltpu.VMEM((B,), jnp.int32),
        pltpu.VMEM((B + 32,), jnp.int32),
        pltpu.VMEM((B + 32,), jnp.int32),
        [
            [pltpu.VMEM((16, 128), jnp.float32) for _ in range(n_val)]
            for _ in range(KR)
        ],
        [pltpu.SemaphoreType.DMA for _ in range(KR)],
        [pltpu.SemaphoreType.DMA for _ in range(KR)],
    ]

    @functools.partial(
        pl.kernel,
        out_type=out_t,
        mesh=_sc_mesh(),
        scratch_types=scratch,
        compiler_params=_sc_params(),
    )
    def k(*args):
        ys = args[:n_val]
        src, dst = args[n_val], args[n_val + 1]
        outs = args[n_val + 2 : 2 * n_val + 2]
        (slabs, zbuf, dv, sv, cd, cs, rowr, semg, sems) = args[
            2 * n_val + 2 :
        ]
        core = lax.axis_index("c")
        sid = lax.axis_index("s")

        def zb(i, _):
            zbuf[i // 8, pl.ds((i % 8) * 16, 16)] = jnp.zeros((16,), jnp.float32)
            return 0

        lax.fori_loop(0, 32 * 8, zb, 0)

        def chunk_body(ci, _):
            chunk = ci * 2 + core
            lo = chunk * C

            def zc(i, _):
                for v in range(n_val):
                    pltpu.sync_copy(
                        zbuf, slabs[v].at[pl.ds(sid * ctile + i * 32, 32)]
                    )
                return 0

            lax.fori_loop(0, ctile // 32, zc, 0)

            @pl.when(sid == 0)
            def _():
                for v in range(n_val):
                    pltpu.sync_copy(
                        zbuf.at[pl.ds(0, 16)], slabs[v].at[pl.ds(C, 16)]
                    )

            plsc.subcore_barrier()

            def blk_body(b, _):
                base = sid * ept + b * B
                pltpu.sync_copy(dst.at[pl.ds(base, B)], dv)
                pltpu.sync_copy(src.at[pl.ds(base, B)], sv)

                def g_body(g, cnt):
                    d16 = dv[pl.ds(g * 16, 16)]
                    s16 = sv[pl.ds(g * 16, 16)]
                    m = (d16 >= lo) & (d16 < lo + C)
                    plsc.store_compressed(cd.at[pl.ds(cnt, 16)], d16 - lo, mask=m)
                    plsc.store_compressed(cs.at[pl.ds(cnt, 16)], s16, mask=m)
                    return cnt + jnp.sum(m.astype(jnp.int32))

                cnt = lax.fori_loop(0, B // 16, g_body, 0)
                cd[pl.ds(cnt, 16)] = jnp.full((16,), C, jnp.int32)
                cs[pl.ds(cnt, 16)] = jnp.zeros((16,), jnp.int32)
                ngv = (cnt + 15) // 16
                nround = (ngv + KR - 1) // KR

                def r_body(i, _):
                    g0 = i * KR
                    for r in range(KR):

                        @pl.when(g0 + r < ngv)
                        def _():
                            d16p = cd[pl.ds((g0 + r) * 16, 16)]
                            @pl.when(g0 + r - KR >= 0)
                            def _():
                                for v in range(n_val):
                                    pltpu.make_async_copy(
                                        rowr[r][v], slabs[v].at[d16p], sems[r]
                                    ).wait()

                            s16 = cs[pl.ds((g0 + r) * 16, 16)]
                            for v in range(n_val):
                                pltpu.async_copy(
                                    ys[v].at[s16], rowr[r][v], semg[r]
                                )
                    for r in range(KR):

                        @pl.when(g0 + r < ngv)
                        def _():
                            d16 = cd[pl.ds((g0 + r) * 16, 16)]
                            for v in range(n_val):
                                pltpu.make_async_copy(
                                    ys[v].at[d16], rowr[r][v], semg[r]
                                ).wait()
                                pltpu.async_copy(
                                    rowr[r][v], slabs[v].at[d16], sems[r],
                                    add=True,
                                )

                    return 0

                lax.fori_loop(0, nround, r_body, 0)
                for r in range(KR):

                    @pl.when(r < ngv)
                    def _():
                        d16 = cd[pl.ds(r * 16, 16)]
                        for v in range(n_val):
                            pltpu.make_async_copy(
                                rowr[r][v], slabs[v].at[d16], sems[r]
                            ).wait()

                return 0

            lax.fori_loop(0, nblk, blk_body, 0)
            plsc.subcore_barrier()
            for v in range(n_val):
                pltpu.sync_copy(
                    slabs[v].at[pl.ds(sid * ctile, ctile)],
                    outs[v].at[pl.ds(chunk * C + sid * ctile, ctile)],
                )
            plsc.subcore_barrier()
            return 0

        lax.fori_loop(0, kpc, chunk_body, 0)

    return k


def _make_prescale(N, BT):
    grid = (pl.cdiv(N, BT),)

    def body(deg_ref, x_ref, y_ref, d_ref):
        d = lax.rsqrt(deg_ref[...] + 1.0)
        d_ref[...] = d
        y_ref[...] = x_ref[...] * d.reshape(BT, 1)

    return pl.pallas_call(
        body,
        grid=grid,
        in_specs=[
            pl.BlockSpec((1, BT), lambda i: (0, i)),
            pl.BlockSpec((BT, 128), lambda i: (i, 0)),
        ],
        out_specs=[
            pl.BlockSpec((BT, 128), lambda i: (i, 0)),
            pl.BlockSpec((1, BT), lambda i: (0, i)),
        ],
        out_shape=[
            jax.ShapeDtypeStruct((N, 128), jnp.float32),
            jax.ShapeDtypeStruct((1, N), jnp.float32),
        ],
    )


def _make_layer0(N, BT):
    grid = (pl.cdiv(N, BT),)

    def body(s_ref, y_ref, d_ref, w_ref, b_ref, a_ref, b2_ref):
        d = d_ref[...].reshape(BT, 1)
        a = (s_ref[...] + y_ref[...]) * d
        z = jnp.maximum(
            jnp.dot(a, w_ref[...], preferred_element_type=jnp.float32)
            + b_ref[...],
            0.0,
        )
        y1 = z * d
        a_ref[...] = y1[:, :128]
        b2_ref[...] = y1[:, 128:]

    return pl.pallas_call(
        body,
        grid=grid,
        in_specs=[
            pl.BlockSpec((BT, 128), lambda i: (i, 0)),
            pl.BlockSpec((BT, 128), lambda i: (i, 0)),
            pl.BlockSpec((1, BT), lambda i: (0, i)),
            pl.BlockSpec((128, 256), lambda i: (0, 0)),
            pl.BlockSpec((1, 256), lambda i: (0, 0)),
        ],
        out_specs=[
            pl.BlockSpec((BT, 128), lambda i: (i, 0)),
            pl.BlockSpec((BT, 128), lambda i: (i, 0)),
        ],
        out_shape=[
            jax.ShapeDtypeStruct((N, 128), jnp.float32),
            jax.ShapeDtypeStruct((N, 128), jnp.float32),
        ],
    )


def _make_layer12(N, BT):
    grid = (pl.cdiv(N, BT),)

    def body(sa, sb, ya, yb, d_ref, w1, b1, w2, out):
        d = d_ref[...].reshape(BT, 1)
        a = (
            jnp.concatenate([sa[...] + ya[...], sb[...] + yb[...]], axis=1) * d
        )
        h = jnp.maximum(
            jnp.dot(a, w1[...], preferred_element_type=jnp.float32) + b1[...],
            0.0,
        )
        out[...] = jnp.dot(h, w2[...], preferred_element_type=jnp.float32) * d

    return pl.pallas_call(
        body,
        grid=grid,
        in_specs=[
            pl.BlockSpec((BT, 128), lambda i: (i, 0)),
            pl.BlockSpec((BT, 128), lambda i: (i, 0)),
            pl.BlockSpec((BT, 128), lambda i: (i, 0)),
            pl.BlockSpec((BT, 128), lambda i: (i, 0)),
            pl.BlockSpec((1, BT), lambda i: (0, i)),
            pl.BlockSpec((256, 512), lambda i: (0, 0)),
            pl.BlockSpec((1, 512), lambda i: (0, 0)),
            pl.BlockSpec((512, 128), lambda i: (0, 0)),
        ],
        out_specs=pl.BlockSpec((BT, 128), lambda i: (i, 0)),
        out_shape=jax.ShapeDtypeStruct((N, 128), jnp.float32),
    )


def _make_final(N, BT):
    grid = (pl.cdiv(N, BT),)

    def body(s_ref, y_ref, d_ref, b_ref, out):
        d = d_ref[...].reshape(BT, 1)
        out[...] = (s_ref[...] + y_ref[...]) * d + b_ref[...]

    return pl.pallas_call(
        body,
        grid=grid,
        in_specs=[
            pl.BlockSpec((BT, 128), lambda i: (i, 0)),
            pl.BlockSpec((BT, 128), lambda i: (i, 0)),
            pl.BlockSpec((1, BT), lambda i: (0, i)),
            pl.BlockSpec((1, 128), lambda i: (0, 0)),
        ],
        out_specs=pl.BlockSpec((BT, 128), lambda i: (i, 0)),
        out_shape=jax.ShapeDtypeStruct((N, 128), jnp.float32),
    )


def kernel(x, edge_index, W0, b0, W1, b1, W2, b2):
    N = x.shape[0]
    E = edge_index.shape[1]

    src = edge_index[0].astype(jnp.int32)
    dst = edge_index[1].astype(jnp.int32)
    EP = pl.cdiv(E, 64000) * 64000
    if EP != E:
        pad = jnp.full((EP - E,), SENTINEL, jnp.int32)
        src = jnp.concatenate([src, jnp.zeros((EP - E,), jnp.int32)])
        dst = jnp.concatenate([dst, pad])

    NHR = pl.cdiv(N, 128 * 128) * 128
    degp = _make_deg_kernel(EP, NHR)(dst)
    dd = degp.reshape(2, NHR * 128)
    deg = (dd[0, :N] + dd[1, :N]).reshape(1, N)

    BT = 2048
    y0, dcol = _make_prescale(N, BT)(deg, x)

    C1, K1 = 12288, 2 * pl.cdiv(N, 2 * 12288)
    scat1 = _make_scatter_kernel(N, EP, C1, K1, 1)
    (s0,) = scat1(y0, src, dst)
    y1a, y1b = _make_layer0(N, BT)(s0, y0, dcol, W0, b0.reshape(1, -1))

    C2, K2 = 5632, 2 * pl.cdiv(N, 2 * 5632)
    scat2 = _make_scatter_kernel(N, EP, C2, K2, 2)
    s1a, s1b = scat2(y1a, y1b, src, dst)
    y2 = _make_layer12(N, BT)(
        s1a, s1b, y1a, y1b, dcol, W1, b1.reshape(1, -1), W2
    )

    (s2,) = scat1(y2, src, dst)
    out = _make_final(N, BT)(s2, y2, dcol, b2.reshape(1, -1))
    return out
